# baseline (speedup 1.0000x reference)
import jax
import jax.numpy as jnp
import numpy as np
from jax import lax
from jax.experimental import pallas as pl
from jax.experimental.pallas import tpu as pltpu
from jax.experimental.pallas import tpu_sc as plsc

_N = 50000
_E = 1600000
_G = 64
_IN_F = 3
_HID = 16
_OUT_F = 32
_NCLS = 10
_K = 4
_EPS = 1e-5

_NC = 2
_NS = 16
_NW = _NC * _NS
_EPW = _E // _NW
_C = 80
_NCH = _EPW // _C

_RPT = 3200
_RTL = _N - _RPT * (_NS - 1)

_NB = 2000
_NBLK = _N // _NB
_EBB = 3200
_EBLK = _E // _EBB


def _xw_body(x_ref, w_ref, y_ref):
    y_ref[...] = jnp.dot(x_ref[...], w_ref[...],
                         preferred_element_type=jnp.float32
                         ).astype(jnp.bfloat16)


_xw_call = pl.pallas_call(
    _xw_body,
    grid=(_NBLK,),
    in_specs=[pl.BlockSpec((_NB, _IN_F), lambda i: (i, 0)),
              pl.BlockSpec((_IN_F, _K * _HID), lambda i: (0, 0))],
    out_specs=pl.BlockSpec((_NB, _K * _HID), lambda i: (i, 0)),
    out_shape=jax.ShapeDtypeStruct((_N, _K * _HID), jnp.bfloat16),
)


def _bn_scale_shift(st_ref, g_ref, b_ref):
    mean = st_ref[0:1, :] * (1.0 / _N)
    var = st_ref[1:2, :] * (1.0 / _N) - mean * mean
    scale = g_ref[...] * lax.rsqrt(var + _EPS)
    shift = b_ref[...] - mean * scale
    return scale, shift


def _elu_mean(acc_ref, deg_ref):
    acc = acc_ref[0] + acc_ref[1]
    deg = deg_ref[0, 0, 0, :] + deg_ref[1, 0, 0, :]
    deg = jnp.maximum(deg, 1.0)
    h = acc / deg[:, None]
    return jnp.where(h > 0.0, h, jnp.exp(h) - 1.0)


def _mid_body(acc_ref, deg_ref, g_ref, b_ref, w_ref, y_ref, h_sc, st_sc):
    p = pl.program_id(0)
    i = pl.program_id(1)

    @pl.when(p == 0)
    def _():
        h = _elu_mean(acc_ref, deg_ref)
        h_sc[pl.ds(i * _NB, _NB), :] = h

        @pl.when(i == 0)
        def _():
            st_sc[...] = jnp.zeros((2, _HID), jnp.float32)

        st_sc[0:1, :] += jnp.sum(h, axis=0, keepdims=True)
        st_sc[1:2, :] += jnp.sum(h * h, axis=0, keepdims=True)

    @pl.when(p == 1)
    def _():
        scale, shift = _bn_scale_shift(st_sc, g_ref, b_ref)
        hb = h_sc[pl.ds(i * _NB, _NB), :] * scale + shift
        y_ref[...] = jnp.dot(hb, w_ref[...],
                             preferred_element_type=jnp.float32
                             ).astype(jnp.bfloat16)


_mid_call = pl.pallas_call(
    _mid_body,
    grid=(2, _NBLK),
    in_specs=[pl.BlockSpec((2, _NB, _HID), lambda p, i: (0, i, 0)),
              pl.BlockSpec((2, 1, 1, _NB), lambda p, i: (0, i, 0, 0)),
              pl.BlockSpec((1, _HID), lambda p, i: (0, 0)),
              pl.BlockSpec((1, _HID), lambda p, i: (0, 0)),
              pl.BlockSpec((_HID, _K * _OUT_F), lambda p, i: (0, 0))],
    out_specs=pl.BlockSpec((_NB, _K * _OUT_F), lambda p, i: (i, 0)),
    out_shape=jax.ShapeDtypeStruct((_N, _K * _OUT_F), jnp.bfloat16),
    scratch_shapes=[pltpu.VMEM((_N, _HID), jnp.float32),
                    pltpu.VMEM((2, _HID), jnp.float32)],
)


def _fin_body(acc_ref, deg_ref, g_ref, b_ref, batch_ref, wfc_ref, o_ref,
              h_sc, st_sc, ps_ref, cnt_ref):
    p = pl.program_id(0)
    i = pl.program_id(1)

    @pl.when(p == 0)
    def _():
        h = _elu_mean(acc_ref, deg_ref)
        h_sc[pl.ds(i * _NB, _NB), :] = h

        @pl.when(i == 0)
        def _():
            st_sc[...] = jnp.zeros((2, _OUT_F), jnp.float32)

        st_sc[0:1, :] += jnp.sum(h, axis=0, keepdims=True)
        st_sc[1:2, :] += jnp.sum(h * h, axis=0, keepdims=True)

    @pl.when(p == 1)
    def _():
        scale, shift = _bn_scale_shift(st_sc, g_ref, b_ref)
        hb = h_sc[pl.ds(i * _NB, _NB), :] * scale + shift
        bt = batch_ref[0, 0, :]
        oh = (bt[:, None] ==
              lax.broadcasted_iota(jnp.int32, (_NB, _G), 1)
              ).astype(jnp.float32)

        @pl.when(i == 0)
        def _():
            ps_ref[...] = jnp.zeros((_G, _OUT_F), jnp.float32)
            cnt_ref[...] = jnp.zeros((_G, _OUT_F), jnp.float32)

        dn = (((0,), (0,)), ((), ()))
        ps_ref[...] += lax.dot_general(oh, hb, dn,
                                       preferred_element_type=jnp.float32)
        cnt_ref[...] += lax.dot_general(
            oh, jnp.ones((_NB, _OUT_F), jnp.float32), dn,
            preferred_element_type=jnp.float32)

        @pl.when(i == _NBLK - 1)
        def _():
            pooled = ps_ref[...] / jnp.maximum(cnt_ref[...], 1.0)
            o_ref[...] = jnp.dot(pooled, wfc_ref[...],
                                 preferred_element_type=jnp.float32)


_fin_call = pl.pallas_call(
    _fin_body,
    grid=(2, _NBLK),
    in_specs=[pl.BlockSpec((2, _NB, _OUT_F), lambda p, i: (0, i, 0)),
              pl.BlockSpec((2, 1, 1, _NB), lambda p, i: (0, i, 0, 0)),
              pl.BlockSpec((1, _OUT_F), lambda p, i: (0, 0)),
              pl.BlockSpec((1, _OUT_F), lambda p, i: (0, 0)),
              pl.BlockSpec((1, 1, _NB), lambda p, i: (i, 0, 0)),
              pl.BlockSpec((_OUT_F, _NCLS), lambda p, i: (0, 0))],
    out_specs=pl.BlockSpec((_G, _NCLS), lambda p, i: (0, 0)),
    out_shape=jax.ShapeDtypeStruct((_G, _NCLS), jnp.float32),
    scratch_shapes=[pltpu.VMEM((_N, _OUT_F), jnp.float32),
                    pltpu.VMEM((2, _OUT_F), jnp.float32),
                    pltpu.VMEM((_G, _OUT_F), jnp.float32),
                    pltpu.VMEM((_G, _OUT_F), jnp.float32)],
)


_GD = 3
_MD = 2
_UN = _GD * _MD
_ER = _E // _C
_RPW = _ER // _NW
_NP = (_RPW - 1) // _UN * _UN
_MASK = -65536


def _make_edge_call(F, with_deg):
    KF = _K * F
    mesh = plsc.VectorSubcoreMesh(core_axis_name="c", subcore_axis_name="s",
                                  num_cores=_NC, num_subcores=_NS)
    if with_deg:
        out_type = (jax.ShapeDtypeStruct((_NC, _N, F), jnp.float32),
                    jax.ShapeDtypeStruct((_NC, _N), jnp.float32))
    else:
        out_type = jax.ShapeDtypeStruct((_NC, _N, F), jnp.float32)

    scratch = []
    scratch += [pltpu.VMEM((2, _C), jnp.int32)] * _GD
    scratch += [pltpu.VMEM((2, _C), jnp.float32)] * _GD
    scratch += [pltpu.VMEM((_C, KF // 2), jnp.int32)] * _GD
    scratch += [pltpu.VMEM((_C, F), jnp.float32)] * _MD
    scratch += [pltpu.VMEM((_C,), jnp.int32)] * _MD
    scratch.append(pltpu.VMEM((_C,), jnp.float32))
    scratch.append(pltpu.VMEM_SHARED((_N, F), jnp.float32))
    if with_deg:
        scratch.append(pltpu.VMEM((_C,), jnp.float32))
        scratch.append(pltpu.VMEM_SHARED((_N,), jnp.float32))
    nsem = 2 * _GD + _MD + (_MD if with_deg else 0)
    scratch += [pltpu.SemaphoreType.DMA] * nsem

    def body(*refs):
        if with_deg:
            (ei, eat, y, acc_out, deg_out) = refs[:5]
            rest = refs[5:]
        else:
            (ei, eat, y, acc_out) = refs[:4]
            rest = refs[4:]
        eiv = rest[0:_GD]
        eav = rest[_GD:2 * _GD]
        rows = rest[2 * _GD:3 * _GD]
        msg = rest[3 * _GD:3 * _GD + _MD]
        dstc = rest[3 * _GD + _MD:3 * _GD + 2 * _MD]
        ones_v = rest[3 * _GD + 2 * _MD]
        acc_sh = rest[3 * _GD + 2 * _MD + 1]
        pos = 3 * _GD + 2 * _MD + 2
        if with_deg:
            zdbuf = rest[pos]
            deg_sh = rest[pos + 1]
            pos += 2
        semL = rest[pos:pos + _GD]
        semG = rest[pos + _GD:pos + 2 * _GD]
        semS = rest[pos + 2 * _GD:pos + 2 * _GD + _MD]
        if with_deg:
            semD = rest[pos + 2 * _GD + _MD:pos + 2 * _GD + 2 * _MD]

        cid = lax.axis_index("c")
        sid = lax.axis_index("s")
        wid = sid * _NC + cid

        for k in range(_C // 16):
            ones_v[pl.ds(k * 16, 16)] = jnp.ones((16,), jnp.float32)

        def zrow(r, c):
            for h in range(F // 16):
                msg[0][r, pl.ds(h * 16, 16)] = jnp.zeros((16,), jnp.float32)
            return c

        lax.fori_loop(0, _C, zrow, 0)
        if with_deg:
            for k in range(_C // 16):
                zdbuf[pl.ds(k * 16, 16)] = jnp.zeros((16,), jnp.float32)
        s0 = sid * _RPT
        nz = jnp.where(sid < _NS - 1, _RPT // _C, _RTL // _C)

        def zcopy(q, c):
            pltpu.sync_copy(msg[0], acc_sh.at[pl.ds(s0 + q * _C, _C)])
            if with_deg:
                pltpu.sync_copy(zdbuf, deg_sh.at[pl.ds(s0 + q * _C, _C)])
            return c

        lax.fori_loop(0, nz, zcopy, 0)

        plsc.subcore_barrier()

        rbase = wid * _RPW

        def lin_issue(r, b):
            pltpu.async_copy(ei.at[:, r], eiv[b], semL[b])
            pltpu.async_copy(eat.at[:, r], eav[b], semL[b])

        def lin_wait(b):
            pltpu.make_async_copy(ei.at[:, 0], eiv[b], semL[b]).wait()
            pltpu.make_async_copy(eat.at[:, 0], eav[b], semL[b]).wait()

        def g_issue(b):
            pltpu.async_copy(y.at[eiv[b].at[0]], rows[b], semG[b])

        def g_wait(b):
            pltpu.make_async_copy(y.at[eiv[b].at[0]], rows[b], semG[b]).wait()

        def s_issue(m):
            pltpu.async_copy(msg[m], acc_sh.at[dstc[m]], semS[m], add=True)
            if with_deg:
                pltpu.async_copy(ones_v, deg_sh.at[dstc[m]], semD[m],
                                 add=True)

        def s_wait(m):
            pltpu.make_async_copy(msg[m], acc_sh.at[dstc[m]], semS[m]).wait()
            if with_deg:
                pltpu.make_async_copy(ones_v, deg_sh.at[dstc[m]],
                                      semD[m]).wait()

        def compute(b, m):
            def group(g, c2):
                g16 = g * 16
                f0 = jnp.clip(eav[b][0, pl.ds(g16, 16)], 0.0, 1.0)
                f1 = jnp.clip(eav[b][1, pl.ds(g16, 16)], 0.0, 1.0)
                bb3 = f0 * f1
                bb1 = f0 - bb3
                bb2 = f1 - bb3
                bb0 = (1.0 - f0) - bb2
                bb = (bb0, bb1, bb2, bb3)
                for j in range(16):
                    e = g16 + j
                    if F == 16:
                        m0 = None
                        for blk in range(2):
                            w = rows[b][e, pl.ds(blk * 16, 16)]
                            lo = plsc.bitcast(w << 16, jnp.float32)
                            hi = plsc.bitcast(w & _MASK, jnp.float32)
                            t = lo * bb[2 * blk][j] + hi * bb[2 * blk + 1][j]
                            m0 = t if m0 is None else m0 + t
                        msg[m][e, pl.ds(0, 16)] = m0
                    else:
                        m0 = None
                        m1 = None
                        for s in range(4):
                            w = rows[b][e, pl.ds(s * 16, 16)]
                            lo = plsc.bitcast(w << 16, jnp.float32) * bb[s][j]
                            hi = plsc.bitcast(w & _MASK, jnp.float32) * bb[s][j]
                            m0 = lo if m0 is None else m0 + lo
                            m1 = hi if m1 is None else m1 + hi
                        msg[m][e, pl.ds(0, 16)] = m0
                        msg[m][e, pl.ds(16, 16)] = m1
                return c2

            lax.fori_loop(0, _C // 16, group, 0)
            for k in range(_C // 16):
                dstc[m][pl.ds(k * 16, 16)] = eiv[b][1, pl.ds(k * 16, 16)]

        lin_issue(rbase, 0)
        lin_issue(rbase + 1, 1)
        lin_issue(rbase + 2, 2)
        lin_wait(0)
        g_issue(0)
        lin_wait(1)
        g_issue(1)

        def outer(jo, carry):
            for u in range(_UN):
                j = jo * _UN + u
                pg = u % _GD
                pm = u % _MD
                pg2 = (u + 2) % _GD

                @pl.when(j + 2 < _NP)
                def _():
                    lin_wait(pg2)
                    g_issue(pg2)

                g_wait(pg)

                @pl.when(j >= 2)
                def _():
                    s_wait(pm)

                compute(pg, pm)

                @pl.when(j + 3 < _NP)
                def _():
                    lin_issue(rbase + j + 3, pg)

                s_issue(pm)
            return carry

        lax.fori_loop(0, _NP // _UN, outer, 0)

        for m in range(_MD):
            s_wait(m)
        lin_issue(rbase + _RPW - 1, 0)
        lin_wait(0)
        g_issue(0)
        g_wait(0)
        compute(0, 0)
        s_issue(0)
        s_wait(0)

        plsc.subcore_barrier()

        @pl.when(sid < _NS - 1)
        def _():
            s0 = sid * _RPT
            pltpu.sync_copy(acc_sh.at[pl.ds(s0, _RPT)],
                            acc_out.at[cid, pl.ds(s0, _RPT)])
            if with_deg:
                pltpu.sync_copy(deg_sh.at[pl.ds(s0, _RPT)],
                                deg_out.at[cid, pl.ds(s0, _RPT)])

        @pl.when(sid == _NS - 1)
        def _():
            s0 = (_NS - 1) * _RPT
            pltpu.sync_copy(acc_sh.at[pl.ds(s0, _RTL)],
                            acc_out.at[cid, pl.ds(s0, _RTL)])
            if with_deg:
                pltpu.sync_copy(deg_sh.at[pl.ds(s0, _RTL)],
                                deg_out.at[cid, pl.ds(s0, _RTL)])

    return pl.kernel(body, out_type=out_type, mesh=mesh,
                     scratch_types=scratch,
                     compiler_params=pltpu.CompilerParams(
                         use_tc_tiling_on_sc=False,
                         needs_layout_passes=False))


_edge1 = _make_edge_call(_HID, True)
_edge2 = _make_edge_call(_OUT_F, False)


def _tab_perm(kf):
    return np.array([(2 * (j // 32) + (j % 2)) * 16 + (j % 32) // 2
                     for j in range(kf)], dtype=np.int32)


_P64 = _tab_perm(_K * _HID)
_P128 = _tab_perm(_K * _OUT_F)


def kernel(x, edge_index, edge_attr, batch, W1, gamma1, beta1,
           W2, gamma2, beta2, Wfc):
    w1f = W1.transpose(1, 0, 2).reshape(_IN_F, _K * _HID)
    w2f = W2.transpose(1, 0, 2).reshape(_HID, _K * _OUT_F)
    y1 = _xw_call(x, w1f[:, _P64])
    y1i = lax.bitcast_convert_type(
        y1.reshape(_N, _K * _HID // 2, 2), jnp.int32)

    ei3 = edge_index.reshape(2, _ER, _C)
    eat3 = edge_attr.T.reshape(2, _ER, _C)
    acc1p, degp = _edge1(ei3, eat3, y1i)
    degr = degp.reshape(_NC, _NBLK, 1, _NB)

    y2 = _mid_call(acc1p, degr, gamma1.reshape(1, _HID),
                   beta1.reshape(1, _HID), w2f[:, _P128])
    y2i = lax.bitcast_convert_type(
        y2.reshape(_N, _K * _OUT_F // 2, 2), jnp.int32)

    acc2p = _edge2(ei3, eat3, y2i)

    out = _fin_call(acc2p, degr, gamma2.reshape(1, _OUT_F),
                    beta2.reshape(1, _OUT_F),
                    batch.reshape(_NBLK, 1, _NB), Wfc)
    return out


# speedup vs baseline: 1.2125x; 1.2125x over previous
import jax
import jax.numpy as jnp
import numpy as np
from jax import lax
from jax.experimental import pallas as pl
from jax.experimental.pallas import tpu as pltpu
from jax.experimental.pallas import tpu_sc as plsc

_N = 50000
_E = 1600000
_G = 64
_IN_F = 3
_HID = 16
_OUT_F = 32
_NCLS = 10
_K = 4
_EPS = 1e-5

_NC = 2
_NS = 16
_NW = _NC * _NS
_EPW = _E // _NW
_C = 80
_NCH = _EPW // _C

_RPT = 3200
_RTL = _N - _RPT * (_NS - 1)

_NB = 2000
_NBLK = _N // _NB
_EBB = 3200
_EBLK = _E // _EBB


def _bf16_bits(y):
    u = lax.bitcast_convert_type(y, jnp.int32)
    return lax.shift_right_logical(
        u + 0x7FFF + (lax.shift_right_logical(u, 16) & 1), 16)


def _pack_pair(ylo, yhi):
    return _bf16_bits(ylo) | (_bf16_bits(yhi) << 16)


def _xw_body(x_ref, wlo_ref, whi_ref, y_ref):
    x = x_ref[...]
    ylo = jnp.dot(x, wlo_ref[...], preferred_element_type=jnp.float32)
    yhi = jnp.dot(x, whi_ref[...], preferred_element_type=jnp.float32)
    y_ref[...] = _pack_pair(ylo, yhi)


_xw_call = pl.pallas_call(
    _xw_body,
    grid=(_NBLK,),
    in_specs=[pl.BlockSpec((_NB, _IN_F), lambda i: (i, 0)),
              pl.BlockSpec((_IN_F, _K * _HID // 2), lambda i: (0, 0)),
              pl.BlockSpec((_IN_F, _K * _HID // 2), lambda i: (0, 0))],
    out_specs=pl.BlockSpec((_NB, _K * _HID // 2), lambda i: (i, 0)),
    out_shape=jax.ShapeDtypeStruct((_N, _K * _HID // 2), jnp.int32),
)


def _bn_scale_shift(st_ref, g_ref, b_ref):
    mean = st_ref[0:1, :] * (1.0 / _N)
    var = st_ref[1:2, :] * (1.0 / _N) - mean * mean
    scale = g_ref[...] * lax.rsqrt(var + _EPS)
    shift = b_ref[...] - mean * scale
    return scale, shift


def _elu_mean(acc_ref, deg_ref):
    acc = acc_ref[0] + acc_ref[1]
    deg = deg_ref[0, 0, 0, :] + deg_ref[1, 0, 0, :]
    deg = jnp.maximum(deg, 1.0)
    h = acc / deg[:, None]
    return jnp.where(h > 0.0, h, jnp.exp(h) - 1.0)


def _mid_body(acc_ref, deg_ref, g_ref, b_ref, wlo_ref, whi_ref, y_ref,
              h_sc, st_sc):
    p = pl.program_id(0)
    i = pl.program_id(1)

    @pl.when(p == 0)
    def _():
        h = _elu_mean(acc_ref, deg_ref)
        h_sc[pl.ds(i * _NB, _NB), :] = h

        @pl.when(i == 0)
        def _():
            st_sc[...] = jnp.zeros((2, _HID), jnp.float32)

        st_sc[0:1, :] += jnp.sum(h, axis=0, keepdims=True)
        st_sc[1:2, :] += jnp.sum(h * h, axis=0, keepdims=True)

    @pl.when(p == 1)
    def _():
        scale, shift = _bn_scale_shift(st_sc, g_ref, b_ref)
        hb = h_sc[pl.ds(i * _NB, _NB), :] * scale + shift
        ylo = jnp.dot(hb, wlo_ref[...], preferred_element_type=jnp.float32)
        yhi = jnp.dot(hb, whi_ref[...], preferred_element_type=jnp.float32)
        y_ref[...] = _pack_pair(ylo, yhi)


_mid_call = pl.pallas_call(
    _mid_body,
    grid=(2, _NBLK),
    in_specs=[pl.BlockSpec((2, _NB, _HID), lambda p, i: (0, i, 0)),
              pl.BlockSpec((2, 1, 1, _NB), lambda p, i: (0, i, 0, 0)),
              pl.BlockSpec((1, _HID), lambda p, i: (0, 0)),
              pl.BlockSpec((1, _HID), lambda p, i: (0, 0)),
              pl.BlockSpec((_HID, _K * _OUT_F // 2), lambda p, i: (0, 0)),
              pl.BlockSpec((_HID, _K * _OUT_F // 2), lambda p, i: (0, 0))],
    out_specs=pl.BlockSpec((_NB, _K * _OUT_F // 2), lambda p, i: (i, 0)),
    out_shape=jax.ShapeDtypeStruct((_N, _K * _OUT_F // 2), jnp.int32),
    scratch_shapes=[pltpu.VMEM((_N, _HID), jnp.float32),
                    pltpu.VMEM((2, _HID), jnp.float32)],
)


def _fin_body(acc_ref, deg_ref, g_ref, b_ref, batch_ref, wfc_ref, o_ref,
              h_sc, st_sc, ps_ref, cnt_ref):
    p = pl.program_id(0)
    i = pl.program_id(1)

    @pl.when(p == 0)
    def _():
        h = _elu_mean(acc_ref, deg_ref)
        h_sc[pl.ds(i * _NB, _NB), :] = h

        @pl.when(i == 0)
        def _():
            st_sc[...] = jnp.zeros((2, _OUT_F), jnp.float32)

        st_sc[0:1, :] += jnp.sum(h, axis=0, keepdims=True)
        st_sc[1:2, :] += jnp.sum(h * h, axis=0, keepdims=True)

    @pl.when(p == 1)
    def _():
        scale, shift = _bn_scale_shift(st_sc, g_ref, b_ref)
        hb = h_sc[pl.ds(i * _NB, _NB), :] * scale + shift
        bt = batch_ref[0, 0, :]
        oh = (bt[:, None] ==
              lax.broadcasted_iota(jnp.int32, (_NB, _G), 1)
              ).astype(jnp.float32)

        @pl.when(i == 0)
        def _():
            ps_ref[...] = jnp.zeros((_G, _OUT_F), jnp.float32)
            cnt_ref[...] = jnp.zeros((_G, _OUT_F), jnp.float32)

        dn = (((0,), (0,)), ((), ()))
        ps_ref[...] += lax.dot_general(oh, hb, dn,
                                       preferred_element_type=jnp.float32)
        cnt_ref[...] += lax.dot_general(
            oh, jnp.ones((_NB, _OUT_F), jnp.float32), dn,
            preferred_element_type=jnp.float32)

        @pl.when(i == _NBLK - 1)
        def _():
            pooled = ps_ref[...] / jnp.maximum(cnt_ref[...], 1.0)
            o_ref[...] = jnp.dot(pooled, wfc_ref[...],
                                 preferred_element_type=jnp.float32)


_fin_call = pl.pallas_call(
    _fin_body,
    grid=(2, _NBLK),
    in_specs=[pl.BlockSpec((2, _NB, _OUT_F), lambda p, i: (0, i, 0)),
              pl.BlockSpec((2, 1, 1, _NB), lambda p, i: (0, i, 0, 0)),
              pl.BlockSpec((1, _OUT_F), lambda p, i: (0, 0)),
              pl.BlockSpec((1, _OUT_F), lambda p, i: (0, 0)),
              pl.BlockSpec((1, 1, _NB), lambda p, i: (i, 0, 0)),
              pl.BlockSpec((_OUT_F, _NCLS), lambda p, i: (0, 0))],
    out_specs=pl.BlockSpec((_G, _NCLS), lambda p, i: (0, 0)),
    out_shape=jax.ShapeDtypeStruct((_G, _NCLS), jnp.float32),
    scratch_shapes=[pltpu.VMEM((_N, _OUT_F), jnp.float32),
                    pltpu.VMEM((2, _OUT_F), jnp.float32),
                    pltpu.VMEM((_G, _OUT_F), jnp.float32),
                    pltpu.VMEM((_G, _OUT_F), jnp.float32)],
)


_MD = 2
_ER = _E // _C
_RPW = _ER // _NW
_MASK = -65536


def _make_edge_call(F, with_deg, gd):
    KF = _K * F
    un = gd if gd % _MD == 0 else gd * _MD
    if un % _MD:
        un *= _MD
    np_ = (_RPW - 1) // un * un
    mesh = plsc.VectorSubcoreMesh(core_axis_name="c", subcore_axis_name="s",
                                  num_cores=_NC, num_subcores=_NS)
    if with_deg:
        out_type = (jax.ShapeDtypeStruct((_NC, _N, F), jnp.float32),
                    jax.ShapeDtypeStruct((_NC, _N), jnp.float32))
    else:
        out_type = jax.ShapeDtypeStruct((_NC, _N, F), jnp.float32)

    scratch = []
    scratch += [pltpu.VMEM((2, _C), jnp.int32)] * gd
    scratch += [pltpu.VMEM((2, _C), jnp.float32)] * gd
    scratch += [pltpu.VMEM((_C, KF // 2), jnp.int32)] * gd
    scratch += [pltpu.VMEM((_C, F), jnp.float32)] * _MD
    scratch += [pltpu.VMEM((_C,), jnp.int32)] * _MD
    scratch.append(pltpu.VMEM((_C,), jnp.float32))
    scratch.append(pltpu.VMEM_SHARED((_N, F), jnp.float32))
    if with_deg:
        scratch.append(pltpu.VMEM((_C,), jnp.float32))
        scratch.append(pltpu.VMEM_SHARED((_N,), jnp.float32))
    nsem = 2 * gd + _MD + (_MD if with_deg else 0)
    scratch += [pltpu.SemaphoreType.DMA] * nsem

    def body(*refs):
        if with_deg:
            (ei, eat, y, acc_out, deg_out) = refs[:5]
            rest = refs[5:]
        else:
            (ei, eat, y, acc_out) = refs[:4]
            rest = refs[4:]
        eiv = rest[0:gd]
        eav = rest[gd:2 * gd]
        rows = rest[2 * gd:3 * gd]
        msg = rest[3 * gd:3 * gd + _MD]
        dstc = rest[3 * gd + _MD:3 * gd + 2 * _MD]
        ones_v = rest[3 * gd + 2 * _MD]
        acc_sh = rest[3 * gd + 2 * _MD + 1]
        pos = 3 * gd + 2 * _MD + 2
        if with_deg:
            zdbuf = rest[pos]
            deg_sh = rest[pos + 1]
            pos += 2
        semL = rest[pos:pos + gd]
        semG = rest[pos + gd:pos + 2 * gd]
        semS = rest[pos + 2 * gd:pos + 2 * gd + _MD]
        if with_deg:
            semD = rest[pos + 2 * gd + _MD:pos + 2 * gd + 2 * _MD]

        cid = lax.axis_index("c")
        sid = lax.axis_index("s")
        wid = sid * _NC + cid

        for k in range(_C // 16):
            ones_v[pl.ds(k * 16, 16)] = jnp.ones((16,), jnp.float32)

        def zrow(r, c):
            for h in range(F // 16):
                msg[0][r, pl.ds(h * 16, 16)] = jnp.zeros((16,), jnp.float32)
            return c

        lax.fori_loop(0, _C, zrow, 0)
        if with_deg:
            for k in range(_C // 16):
                zdbuf[pl.ds(k * 16, 16)] = jnp.zeros((16,), jnp.float32)
        s0 = sid * _RPT
        nz = jnp.where(sid < _NS - 1, _RPT // _C, _RTL // _C)

        def zcopy(q, c):
            pltpu.sync_copy(msg[0], acc_sh.at[pl.ds(s0 + q * _C, _C)])
            if with_deg:
                pltpu.sync_copy(zdbuf, deg_sh.at[pl.ds(s0 + q * _C, _C)])
            return c

        lax.fori_loop(0, nz, zcopy, 0)

        plsc.subcore_barrier()

        rbase = wid * _RPW

        def lin_issue(r, b):
            pltpu.async_copy(ei.at[:, r], eiv[b], semL[b])
            pltpu.async_copy(eat.at[:, r], eav[b], semL[b])

        def lin_wait(b):
            pltpu.make_async_copy(ei.at[:, 0], eiv[b], semL[b]).wait()
            pltpu.make_async_copy(eat.at[:, 0], eav[b], semL[b]).wait()

        def g_issue(b):
            pltpu.async_copy(y.at[eiv[b].at[0]], rows[b], semG[b])

        def g_wait(b):
            pltpu.make_async_copy(y.at[eiv[b].at[0]], rows[b], semG[b]).wait()

        def s_issue(m):
            pltpu.async_copy(msg[m], acc_sh.at[dstc[m]], semS[m], add=True)
            if with_deg:
                pltpu.async_copy(ones_v, deg_sh.at[dstc[m]], semD[m],
                                 add=True)

        def s_wait(m):
            pltpu.make_async_copy(msg[m], acc_sh.at[dstc[m]], semS[m]).wait()
            if with_deg:
                pltpu.make_async_copy(ones_v, deg_sh.at[dstc[m]],
                                      semD[m]).wait()

        def compute(b, m):
            def group(g, c2):
                g16 = g * 16
                f0 = jnp.clip(eav[b][0, pl.ds(g16, 16)], 0.0, 1.0)
                f1 = jnp.clip(eav[b][1, pl.ds(g16, 16)], 0.0, 1.0)
                bb3 = f0 * f1
                bb1 = f0 - bb3
                bb2 = f1 - bb3
                bb0 = (1.0 - f0) - bb2
                bb = (bb0, bb1, bb2, bb3)
                for j in range(16):
                    e = g16 + j
                    if F == 16:
                        m0 = None
                        for blk in range(2):
                            w = rows[b][e, pl.ds(blk * 16, 16)]
                            lo = plsc.bitcast(w << 16, jnp.float32)
                            hi = plsc.bitcast(w & _MASK, jnp.float32)
                            t = lo * bb[2 * blk][j] + hi * bb[2 * blk + 1][j]
                            m0 = t if m0 is None else m0 + t
                        msg[m][e, pl.ds(0, 16)] = m0
                    else:
                        m0 = None
                        m1 = None
                        for s in range(4):
                            w = rows[b][e, pl.ds(s * 16, 16)]
                            lo = plsc.bitcast(w << 16, jnp.float32) * bb[s][j]
                            hi = plsc.bitcast(w & _MASK, jnp.float32) * bb[s][j]
                            m0 = lo if m0 is None else m0 + lo
                            m1 = hi if m1 is None else m1 + hi
                        msg[m][e, pl.ds(0, 16)] = m0
                        msg[m][e, pl.ds(16, 16)] = m1
                return c2

            lax.fori_loop(0, _C // 16, group, 0)
            for k in range(_C // 16):
                dstc[m][pl.ds(k * 16, 16)] = eiv[b][1, pl.ds(k * 16, 16)]

        for q in range(gd):
            lin_issue(rbase + q, q)
        for q in range(gd - 1):
            lin_wait(q)
            g_issue(q)

        ga = gd - 1

        def outer(jo, carry):
            for u in range(un):
                j = jo * un + u
                pg = u % gd
                pm = u % _MD
                pga = (u + ga) % gd

                @pl.when(j + ga < np_)
                def _():
                    lin_wait(pga)
                    g_issue(pga)

                g_wait(pg)

                @pl.when(j >= 2)
                def _():
                    s_wait(pm)

                compute(pg, pm)

                @pl.when(j + gd < np_)
                def _():
                    lin_issue(rbase + j + gd, pg)

                s_issue(pm)
            return carry

        lax.fori_loop(0, np_ // un, outer, 0)

        for m in range(_MD):
            s_wait(m)
        lin_issue(rbase + _RPW - 1, 0)
        lin_wait(0)
        g_issue(0)
        g_wait(0)
        compute(0, 0)
        s_issue(0)
        s_wait(0)

        plsc.subcore_barrier()

        @pl.when(sid < _NS - 1)
        def _():
            s0 = sid * _RPT
            pltpu.sync_copy(acc_sh.at[pl.ds(s0, _RPT)],
                            acc_out.at[cid, pl.ds(s0, _RPT)])
            if with_deg:
                pltpu.sync_copy(deg_sh.at[pl.ds(s0, _RPT)],
                                deg_out.at[cid, pl.ds(s0, _RPT)])

        @pl.when(sid == _NS - 1)
        def _():
            s0 = (_NS - 1) * _RPT
            pltpu.sync_copy(acc_sh.at[pl.ds(s0, _RTL)],
                            acc_out.at[cid, pl.ds(s0, _RTL)])
            if with_deg:
                pltpu.sync_copy(deg_sh.at[pl.ds(s0, _RTL)],
                                deg_out.at[cid, pl.ds(s0, _RTL)])

    return pl.kernel(body, out_type=out_type, mesh=mesh,
                     scratch_types=scratch,
                     compiler_params=pltpu.CompilerParams(
                         use_tc_tiling_on_sc=False,
                         needs_layout_passes=False))


_edge1 = _make_edge_call(_HID, True, 2)
_edge2 = _make_edge_call(_OUT_F, False, 3)


def _perm_lo(kf):
    return np.array([2 * (k // 16) * 16 + k % 16 for k in range(kf // 2)],
                    dtype=np.int32)


def _perm_hi(kf):
    return np.array([(2 * (k // 16) + 1) * 16 + k % 16
                     for k in range(kf // 2)], dtype=np.int32)


_PL64, _PH64 = _perm_lo(_K * _HID), _perm_hi(_K * _HID)
_PL128, _PH128 = _perm_lo(_K * _OUT_F), _perm_hi(_K * _OUT_F)


def kernel(x, edge_index, edge_attr, batch, W1, gamma1, beta1,
           W2, gamma2, beta2, Wfc):
    w1f = W1.transpose(1, 0, 2).reshape(_IN_F, _K * _HID)
    w2f = W2.transpose(1, 0, 2).reshape(_HID, _K * _OUT_F)
    y1i = _xw_call(x, w1f[:, _PL64], w1f[:, _PH64])

    ei3 = edge_index.reshape(2, _ER, _C)
    eat3 = edge_attr.T.reshape(2, _ER, _C)
    acc1p, degp = _edge1(ei3, eat3, y1i)
    degr = degp.reshape(_NC, _NBLK, 1, _NB)

    y2i = _mid_call(acc1p, degr, gamma1.reshape(1, _HID),
                    beta1.reshape(1, _HID),
                    w2f[:, _PL128], w2f[:, _PH128])

    acc2p = _edge2(ei3, eat3, y2i)

    out = _fin_call(acc2p, degr, gamma2.reshape(1, _OUT_F),
                    beta2.reshape(1, _OUT_F),
                    batch.reshape(_NBLK, 1, _NB), Wfc)
    return out
